# baseline (speedup 1.0000x reference)
import jax
import jax.numpy as jnp
from jax import lax
from jax.experimental import pallas as pl
from jax.experimental.pallas import tpu as pltpu
from jax.experimental.pallas import tpu_sc as plsc

_NC = 2
_NS = 16
_W = 128
_NB = 6
_G = 4
_PAD = 8


def _mm1_body(x_ref, w_ref, b_ref, ol_ref, or_ref):
    h = or_ref.shape[1]
    f2 = h // 2
    y = jnp.dot(x_ref[...], w_ref[...], preferred_element_type=jnp.float32)
    ol_ref[0] = y[:, :f2]
    ol_ref[1] = y[:, f2:h]
    or_ref[...] = y[:, h:] + b_ref[...]


def _mm1(x, w, b2d):
    n = x.shape[0]
    h = w.shape[1] // 2
    return pl.pallas_call(
        _mm1_body,
        out_shape=[jax.ShapeDtypeStruct((2, n, h // 2), jnp.float32),
                   jax.ShapeDtypeStruct((n, h), jnp.float32)],
    )(x, w, b2d)


def _mm2_body(aggs_ref, xr_ref, w_ref, b_ref, ol_ref, or_ref):
    c = or_ref.shape[1]
    c2 = c // 2
    hmat = jnp.maximum(
        jnp.concatenate([aggs_ref[0], aggs_ref[1]], axis=1) + xr_ref[...], 0.0)
    y = jnp.dot(hmat, w_ref[...], preferred_element_type=jnp.float32)
    ol_ref[0] = y[:, :c2]
    ol_ref[1] = y[:, c2:c]
    or_ref[...] = y[:, c:] + b_ref[...]


def _mm2(aggs, xr, w, b2d):
    n = xr.shape[0]
    c = w.shape[1] // 2
    return pl.pallas_call(
        _mm2_body,
        out_shape=[jax.ShapeDtypeStruct((2, n, c // 2), jnp.float32),
                   jax.ShapeDtypeStruct((n, c), jnp.float32)],
    )(aggs, xr, w, b2d)


def _combine_body(aggs_ref, hr_ref, o_ref):
    o_ref[...] = (jnp.concatenate([aggs_ref[0], aggs_ref[1]], axis=1)
                  + hr_ref[...])


def _combine(aggs, hr):
    return pl.pallas_call(
        _combine_body,
        out_shape=jax.ShapeDtypeStruct(hr.shape, jnp.float32),
    )(aggs, hr)


def _seg_sum_sc(feat2, src4, dst3, n):
    f2 = feat2.shape[1]
    cpw = src4.shape[2]
    rpt = n // _NS

    mesh = plsc.VectorSubcoreMesh(
        core_axis_name="c", subcore_axis_name="s",
        num_cores=_NC, num_subcores=_NS)

    def body(feat_hbm, src_hbm, dst_hbm, out_hbm,
             acc, sidx, didx, rbufs, gsems, ssems):
        c = lax.axis_index("c")
        s = lax.axis_index("s")
        r0 = s * rpt

        pltpu.async_copy(src_hbm.at[c, s], sidx, gsems[0])
        pltpu.async_copy(dst_hbm.at[s], didx, gsems[1])

        zv = jnp.zeros((16,), jnp.float32)

        def zb(r, carry):
            for q in range(f2 // 16):
                rbufs[0][r, pl.ds(q * 16, 16)] = zv
            return carry

        lax.fori_loop(0, _W, zb, 0)
        nfull = rpt // _W
        for t in range(nfull):
            pltpu.sync_copy(rbufs[0], acc.at[pl.ds(r0 + t * _W, _W)])
        rem = rpt - nfull * _W
        if rem:
            pltpu.sync_copy(rbufs[0].at[pl.ds(0, rem)],
                            acc.at[pl.ds(r0 + nfull * _W, rem)])

        pltpu.make_async_copy(src_hbm.at[c, s], sidx, gsems[0]).wait()
        pltpu.make_async_copy(dst_hbm.at[s], didx, gsems[1]).wait()

        def gather(k, b):
            pltpu.async_copy(feat_hbm.at[sidx.at[k]], rbufs[b], gsems[b])

        def gwait(b):
            pltpu.make_async_copy(feat_hbm.at[sidx.at[0]], rbufs[b],
                                  gsems[b]).wait()

        def scat(k, b):
            pltpu.async_copy(rbufs[b], acc.at[didx.at[k]], ssems[b],
                             add=True)

        def swait(b):
            pltpu.make_async_copy(rbufs[b], acc.at[didx.at[0]],
                                  ssems[b]).wait()

        for k in range(_G):
            gather(k, k % _NB)
        plsc.subcore_barrier()

        def chunk_step(k, b):
            gwait(b)
            scat(k, b)
            b2 = (b + _G) % _NB

            @pl.when(k + _G < cpw)
            def _():
                @pl.when(k >= _NB - _G)
                def _():
                    swait(b2)

                gather(k + _G, b2)

        def loop_body(i, carry):
            for b in range(_NB):
                chunk_step(i * _NB + b, b)
            return carry

        nloop = cpw // _NB
        lax.fori_loop(0, nloop, loop_body, 0)
        for k in range(nloop * _NB, cpw):
            chunk_step(k, k % _NB)
        for b in range(_NB):
            swait(b)

        plsc.subcore_barrier()
        pltpu.sync_copy(acc.at[pl.ds(r0, rpt)], out_hbm.at[c, s])

    kern = pl.kernel(
        body,
        out_type=jax.ShapeDtypeStruct((_NC, _NS, rpt, f2), jnp.float32),
        mesh=mesh,
        scratch_types=[
            pltpu.VMEM_SHARED((n + _PAD, f2), jnp.float32),
            pltpu.VMEM((cpw, _W), jnp.int32),
            pltpu.VMEM((cpw, _W), jnp.int32),
            [pltpu.VMEM((_W, f2), jnp.float32) for _ in range(_NB)],
            [pltpu.SemaphoreType.DMA for _ in range(_NB)],
            [pltpu.SemaphoreType.DMA for _ in range(_NB)],
        ],
        compiler_params=pltpu.CompilerParams(use_tc_tiling_on_sc=False),
    )
    return kern(feat2, src4, dst3)


def kernel(x, edge_index, W1l, b1, W1r, W2l, b2, W2r):
    n, d = x.shape
    h = W1l.shape[0]
    c = W2l.shape[0]
    e = edge_index.shape[1]

    e_pad = -(-e // (_NS * _W)) * (_NS * _W)
    src = edge_index[0]
    dst = edge_index[1]
    if e_pad != e:
        src = jnp.concatenate([src, jnp.zeros((e_pad - e,), jnp.int32)])
        dst = jnp.concatenate([dst, jnp.full((e_pad - e,), n, jnp.int32)])
    src3 = src.reshape(_NS, -1, _W)
    src4 = jnp.stack([src3, src3 + n])
    dst3 = dst.reshape(_NS, -1, _W)

    wt1 = jnp.concatenate([W1l, W1r], axis=0).T
    xl2, xr = _mm1(x, wt1, b1[None, :])

    aggs1 = _seg_sum_sc(xl2.reshape(2 * n, h // 2), src4, dst3, n)
    aggs1 = aggs1.reshape(2, n, h // 2)

    wt2 = jnp.concatenate([W2l, W2r], axis=0).T
    hl2, hr = _mm2(aggs1, xr, wt2, b2[None, :])

    aggs2 = _seg_sum_sc(hl2.reshape(2 * n, c // 2), src4, dst3, n)
    aggs2 = aggs2.reshape(2, n, c // 2)
    return _combine(aggs2, hr)


# speedup vs baseline: 1.4255x; 1.4255x over previous
import jax
import jax.numpy as jnp
from jax import lax
from jax.experimental import pallas as pl
from jax.experimental.pallas import tpu as pltpu
from jax.experimental.pallas import tpu_sc as plsc

_NC = 2
_NS = 16
_W = 80


def _mm_split_body(x_ref, w_ref, b_ref, ol_ref, or_ref):
    h = ol_ref.shape[1]
    y = jnp.dot(x_ref[...], w_ref[...], preferred_element_type=jnp.float32)
    ol_ref[...] = y[:, :h]
    or_ref[...] = y[:, h:] + b_ref[...]


def _mm_split(x, w, b2d):
    n = x.shape[0]
    k = w.shape[1] // 2
    return pl.pallas_call(
        _mm_split_body,
        out_shape=[jax.ShapeDtypeStruct((n, k), jnp.float32),
                   jax.ShapeDtypeStruct((n, k), jnp.float32)],
    )(x, w, b2d)


def _layer2_body(aggs_ref, xr_ref, w_ref, b_ref, ol_ref, or_ref):
    n = xr_ref.shape[0]
    c = ol_ref.shape[1]
    h = jnp.maximum(aggs_ref[:n, :] + aggs_ref[n:, :] + xr_ref[...], 0.0)
    y = jnp.dot(h, w_ref[...], preferred_element_type=jnp.float32)
    ol_ref[...] = y[:, :c]
    or_ref[...] = y[:, c:] + b_ref[...]


def _layer2(aggs, xr, w, b2d):
    n = xr.shape[0]
    c = w.shape[1] // 2
    return pl.pallas_call(
        _layer2_body,
        out_shape=[jax.ShapeDtypeStruct((n, c), jnp.float32),
                   jax.ShapeDtypeStruct((n, c), jnp.float32)],
    )(aggs, xr, w, b2d)


def _combine_body(aggs_ref, hr_ref, o_ref):
    n = hr_ref.shape[0]
    o_ref[...] = aggs_ref[:n, :] + aggs_ref[n:, :] + hr_ref[...]


def _combine(aggs, hr):
    return pl.pallas_call(
        _combine_body,
        out_shape=jax.ShapeDtypeStruct(hr.shape, jnp.float32),
    )(aggs, hr)


def _seg_sum_sc(feat, ei4, nb, g):
    n, f = feat.shape
    cpw = ei4.shape[2]
    rpt = n // _NS

    mesh = plsc.VectorSubcoreMesh(
        core_axis_name="c", subcore_axis_name="s",
        num_cores=_NC, num_subcores=_NS)

    def body(feat_hbm, ei_hbm, out_hbm, acc, sidx, didx, rbufs, gsems, ssems):
        c = lax.axis_index("c")
        s = lax.axis_index("s")
        wid = s * _NC + c
        r0 = s * rpt

        pltpu.async_copy(ei_hbm.at[0, wid], sidx, gsems[0])
        pltpu.async_copy(ei_hbm.at[1, wid], didx, gsems[1])

        zv = jnp.zeros((16,), jnp.float32)

        def zb(r, carry):
            for q in range(f // 16):
                rbufs[0][r, pl.ds(q * 16, 16)] = zv
            return carry

        lax.fori_loop(0, _W, zb, 0)
        nfull = rpt // _W
        for t in range(nfull):
            pltpu.sync_copy(rbufs[0], acc.at[pl.ds(r0 + t * _W, _W)])
        rem = rpt - nfull * _W
        if rem:
            pltpu.sync_copy(rbufs[0].at[pl.ds(0, rem)],
                            acc.at[pl.ds(r0 + nfull * _W, rem)])

        pltpu.make_async_copy(ei_hbm.at[0, wid], sidx, gsems[0]).wait()
        pltpu.make_async_copy(ei_hbm.at[1, wid], didx, gsems[1]).wait()

        def gather(k, b):
            pltpu.async_copy(feat_hbm.at[sidx.at[k]], rbufs[b], gsems[b])

        def gwait(b):
            pltpu.make_async_copy(feat_hbm.at[sidx.at[0]], rbufs[b],
                                  gsems[b]).wait()

        def scat(k, b):
            pltpu.async_copy(rbufs[b], acc.at[didx.at[k]], ssems[b],
                             add=True)

        def swait(b):
            pltpu.make_async_copy(rbufs[b], acc.at[didx.at[0]],
                                  ssems[b]).wait()

        for k in range(g):
            gather(k, k % nb)
        plsc.subcore_barrier()

        def chunk_step(k, b):
            gwait(b)
            scat(k, b)
            b2 = (b + g) % nb

            @pl.when(k + g < cpw)
            def _():
                @pl.when(k >= nb - g)
                def _():
                    swait(b2)

                gather(k + g, b2)

        def loop_body(i, carry):
            for b in range(nb):
                chunk_step(i * nb + b, b)
            return carry

        nloop = cpw // nb
        lax.fori_loop(0, nloop, loop_body, 0)
        for k in range(nloop * nb, cpw):
            chunk_step(k, k % nb)
        for b in range(nb):
            swait(b)

        plsc.subcore_barrier()
        pltpu.sync_copy(acc.at[pl.ds(r0, rpt)], out_hbm.at[c, s])

    kern = pl.kernel(
        body,
        out_type=jax.ShapeDtypeStruct((_NC, _NS, rpt, f), jnp.float32),
        mesh=mesh,
        scratch_types=[
            pltpu.VMEM_SHARED((n, f), jnp.float32),
            pltpu.VMEM((cpw, _W), jnp.int32),
            pltpu.VMEM((cpw, _W), jnp.int32),
            [pltpu.VMEM((_W, f), jnp.float32) for _ in range(nb)],
            [pltpu.SemaphoreType.DMA for _ in range(nb)],
            [pltpu.SemaphoreType.DMA for _ in range(nb)],
        ],
        compiler_params=pltpu.CompilerParams(use_tc_tiling_on_sc=False),
    )
    return kern(feat, ei4)


def kernel(x, edge_index, W1l, b1, W1r, W2l, b2, W2r):
    n, d = x.shape
    h = W1l.shape[0]
    c = W2l.shape[0]

    nw = _NC * _NS
    ei4 = edge_index.reshape(2, nw, -1, _W)

    wt1 = jnp.concatenate([W1l, W1r], axis=0).T
    xl, xr = _mm_split(x, wt1, b1[None, :])

    aggs1 = _seg_sum_sc(xl, ei4, nb=3, g=2).reshape(2 * n, h)

    wt2 = jnp.concatenate([W2l, W2r], axis=0).T
    hl, hr = _layer2(aggs1, xr, wt2, b2[None, :])

    aggs2 = _seg_sum_sc(hl, ei4, nb=6, g=4).reshape(2 * n, c)
    return _combine(aggs2, hr)
